# Initial kernel scaffold; baseline (speedup 1.0000x reference)
import jax
import jax.numpy as jnp
from jax.experimental import pallas as pl


def kernel(x, edge_index, W, b):
    raise NotImplementedError("write your pallas kernel here")


# speedup vs baseline: 20.0379x; 20.0379x over previous
import functools

import jax
import jax.numpy as jnp
from jax import lax
from jax.experimental import pallas as pl
from jax.experimental.pallas import tpu as pltpu
from jax.experimental.pallas import tpu_sc as plsc

N = 10000
E = 320000
D = 128

NC = 2
NS = 16
L = 16
NW = NC * NS

ROWS_PER_TILE = 640
N_PAD = NS * ROWS_PER_TILE
TRASH = N

CH = 128
EPW = E // NW
NCHUNK = (EPW + CH - 1) // CH
EPW_PAD = NCHUNK * CH


def _sc_degree(dst_pad):
    mesh = plsc.VectorSubcoreMesh(core_axis_name="c", subcore_axis_name="s")

    @functools.partial(
        pl.kernel,
        out_type=jax.ShapeDtypeStruct((NC, N_PAD), jnp.float32),
        mesh=mesh,
        scratch_types=[
            pltpu.VMEM((NCHUNK, CH), jnp.int32),
            pltpu.VMEM((CH,), jnp.float32),
            pltpu.VMEM((ROWS_PER_TILE,), jnp.float32),
            pltpu.VMEM_SHARED((N_PAD,), jnp.float32),
        ],
    )
    def deg_kernel(dst_hbm, out_hbm, idx_v, ones_v, zero_v, deg_sh):
        c = lax.axis_index("c")
        s = lax.axis_index("s")
        w = s * NC + c

        def fill_ones(i, _):
            ones_v[pl.ds(i * L, L)] = jnp.full((L,), 1.0, jnp.float32)
            return 0

        lax.fori_loop(0, CH // L, fill_ones, 0)

        def fill_zero(i, _):
            zero_v[pl.ds(i * L, L)] = jnp.zeros((L,), jnp.float32)
            return 0

        lax.fori_loop(0, ROWS_PER_TILE // L, fill_zero, 0)

        pltpu.sync_copy(zero_v, deg_sh.at[pl.ds(s * ROWS_PER_TILE, ROWS_PER_TILE)])
        plsc.subcore_barrier()

        pltpu.sync_copy(dst_hbm.at[w], idx_v)

        def body(j, _):
            pltpu.sync_copy(ones_v, deg_sh.at[idx_v.at[j]], add=True)
            return 0

        lax.fori_loop(0, NCHUNK, body, 0)
        plsc.subcore_barrier()

        sl = pl.ds(s * ROWS_PER_TILE, ROWS_PER_TILE)
        pltpu.sync_copy(deg_sh.at[sl], out_hbm.at[c, sl])

    return deg_kernel(dst_pad)


def _tc_g(x, W, deg_t):
    R = 400

    def gk(x_ref, w_ref, deg_ref, g_ref):
        h = lax.dot_general(
            x_ref[...], w_ref[...], (((1,), (1,)), ((), ())),
            preferred_element_type=jnp.float32,
        )
        d = deg_ref[...]
        dis = lax.rsqrt(d[:, 0:1] + d[:, 1:2] + 1.0)
        g_ref[...] = h * dis

    return pl.pallas_call(
        gk,
        grid=(N // R,),
        in_specs=[
            pl.BlockSpec((R, D), lambda i: (i, 0)),
            pl.BlockSpec((D, D), lambda i: (0, 0)),
            pl.BlockSpec((R, NC), lambda i: (i, 0)),
        ],
        out_specs=pl.BlockSpec((R, D), lambda i: (i, 0)),
        out_shape=jax.ShapeDtypeStruct((N, D), jnp.float32),
    )(x, W, deg_t)


def _sc_agg(g, src_pad, dst_pad):
    mesh = plsc.VectorSubcoreMesh(core_axis_name="c", subcore_axis_name="s")

    @functools.partial(
        pl.kernel,
        out_type=jax.ShapeDtypeStruct((NC, N_PAD, D), jnp.float32),
        mesh=mesh,
        scratch_types=[
            pltpu.VMEM((NCHUNK, CH), jnp.int32),
            pltpu.VMEM((NCHUNK, CH), jnp.int32),
            pltpu.VMEM((CH, D), jnp.float32),
            pltpu.VMEM_SHARED((N_PAD, D), jnp.float32),
            pltpu.SemaphoreType.DMA,
        ],
    )
    def agg_kernel(g_hbm, src_hbm, dst_hbm, out_hbm, si_v, di_v, rows_v,
                   acc_sh, sem):
        c = lax.axis_index("c")
        s = lax.axis_index("s")
        w = s * NC + c

        def zrow(i, _):
            def zlane(k, _):
                rows_v[i, pl.ds(k * L, L)] = jnp.zeros((L,), jnp.float32)
                return 0

            lax.fori_loop(0, D // L, zlane, 0)
            return 0

        lax.fori_loop(0, CH, zrow, 0)

        def zcopy(i, _):
            pltpu.sync_copy(rows_v, acc_sh.at[pl.ds(s * ROWS_PER_TILE + i * CH, CH)])
            return 0

        lax.fori_loop(0, ROWS_PER_TILE // CH, zcopy, 0)
        plsc.subcore_barrier()

        pltpu.sync_copy(src_hbm.at[w], si_v)
        pltpu.sync_copy(dst_hbm.at[w], di_v)

        def body(j, _):
            pltpu.async_copy(g_hbm.at[si_v.at[j]], rows_v, sem).wait()
            pltpu.sync_copy(rows_v, acc_sh.at[di_v.at[j]], add=True)
            return 0

        lax.fori_loop(0, NCHUNK, body, 0)
        plsc.subcore_barrier()

        def flush(i, _):
            base = s * ROWS_PER_TILE + i * CH
            pltpu.sync_copy(acc_sh.at[pl.ds(base, CH)], out_hbm.at[c, pl.ds(base, CH)])
            return 0

        lax.fori_loop(0, ROWS_PER_TILE // CH, flush, 0)

    return agg_kernel(g, src_pad, dst_pad)


def _tc_final(acc, g, deg_t, b2):
    R = 400

    def fk(acc_ref, g_ref, deg_ref, b_ref, o_ref):
        a = acc_ref[0] + acc_ref[1] + g_ref[...]
        d = deg_ref[...]
        dis = lax.rsqrt(d[:, 0:1] + d[:, 1:2] + 1.0)
        o_ref[...] = a * dis + b_ref[...]

    return pl.pallas_call(
        fk,
        grid=(N // R,),
        in_specs=[
            pl.BlockSpec((NC, R, D), lambda i: (0, i, 0)),
            pl.BlockSpec((R, D), lambda i: (i, 0)),
            pl.BlockSpec((R, NC), lambda i: (i, 0)),
            pl.BlockSpec((1, D), lambda i: (0, 0)),
        ],
        out_specs=pl.BlockSpec((R, D), lambda i: (i, 0)),
        out_shape=jax.ShapeDtypeStruct((N, D), jnp.float32),
    )(acc, g, deg_t, b2)


def kernel(x, edge_index, W, b):
    src = edge_index[0].astype(jnp.int32).reshape(NW, EPW)
    dst = edge_index[1].astype(jnp.int32).reshape(NW, EPW)
    pad = EPW_PAD - EPW
    src_p = jnp.pad(src, ((0, 0), (0, pad)), constant_values=0).reshape(
        NW, NCHUNK, CH)
    dst_p = jnp.pad(dst, ((0, 0), (0, pad)), constant_values=TRASH).reshape(
        NW, NCHUNK, CH)

    deg = _sc_degree(dst_p)
    deg_t = deg.T
    g = _tc_g(x, W, deg_t)
    acc = _sc_agg(g, src_p, dst_p)
    return _tc_final(acc, g, deg_t, b.reshape(1, D))
